# Initial kernel scaffold; baseline (speedup 1.0000x reference)
import jax
import jax.numpy as jnp
from jax.experimental import pallas as pl


def kernel(t, state, ind, omegas, w):
    raise NotImplementedError("write your pallas kernel here")


# speedup vs baseline: 7.3468x; 7.3468x over previous
import functools

import jax
import jax.numpy as jnp
from jax import lax
from jax.experimental import pallas as pl
from jax.experimental.pallas import tpu as pltpu
from jax.experimental.pallas import tpu_sc as plsc

_EPS = 0.1
_GDN = lax.GatherDimensionNumbers(
    offset_dims=(), collapsed_slice_dims=(0,), start_index_map=(0,))


def _shuffle(v, idx):
    return lax.gather(v, idx[:, None], _GDN, slice_sizes=(1,),
                      mode=lax.GatherScatterMode.PROMISE_IN_BOUNDS)
_NC = 2
_NS = 16
_LANES = 16
_CHUNK = 512
_GRP = _CHUNK // 128


def _tc_norm_omega(state, omegas):
    N, D = state.shape
    BN = 1000

    def body(st_ref, om_ref, sn_ref, y_ref, s_ref):
        st = st_ref[...]
        inv = lax.rsqrt(jnp.sum(st * st, axis=1, keepdims=True))
        sn = st * inv
        sn_ref[...] = sn
        om = om_ref[...]
        t1 = jnp.sum(om * sn[:, None, :], axis=2)
        t2 = jnp.sum(om * sn[:, :, None], axis=1)
        y_ref[...] = 0.5 * (t1 - t2)

        @pl.when(pl.program_id(0) == 0)
        def _():
            s_ref[...] = jnp.zeros_like(s_ref)

        s_ref[...] += jnp.sum(om, axis=0)

    return pl.pallas_call(
        body,
        grid=(N // BN,),
        in_specs=[
            pl.BlockSpec((BN, D), lambda i: (i, 0)),
            pl.BlockSpec((BN, D, D), lambda i: (i, 0, 0)),
        ],
        out_specs=[
            pl.BlockSpec((BN, D), lambda i: (i, 0)),
            pl.BlockSpec((BN, D), lambda i: (i, 0)),
            pl.BlockSpec((D, D), lambda i: (0, 0)),
        ],
        out_shape=[
            jax.ShapeDtypeStruct((N, D), jnp.float32),
            jax.ShapeDtypeStruct((N, D), jnp.float32),
            jax.ShapeDtypeStruct((D, D), jnp.float32),
        ],
    )(state, omegas)


def _sc_edges(sn_pad, src2d, dst2d, wrep, zeros_pad):
    NP, D = sn_pad.shape
    rows_total = src2d.shape[0]
    e_pad = rows_total * 128
    nw = _NC * _NS
    per_w = e_pad // nw
    n_chunks = per_w // _CHUNK
    stripe = NP // _NS
    K = wrep.shape[0] // _LANES

    mesh = plsc.VectorSubcoreMesh(
        core_axis_name="c", subcore_axis_name="s",
        num_cores=_NC, num_subcores=_NS)

    @functools.partial(
        pl.kernel,
        out_type=jax.ShapeDtypeStruct((_NC, NP, D), jnp.float32),
        mesh=mesh,
        compiler_params=pltpu.CompilerParams(use_tc_tiling_on_sc=False),
        scratch_types=[
            pltpu.VMEM((_GRP, 128), jnp.int32),
            pltpu.VMEM((_GRP, 128), jnp.int32),
            pltpu.VMEM((_CHUNK, D), jnp.float32),
            pltpu.VMEM((_CHUNK, D), jnp.float32),
            pltpu.VMEM((K * _LANES,), jnp.float32),
            pltpu.VMEM_SHARED((NP, D), jnp.float32),
            pltpu.SemaphoreType.DMA,
            pltpu.SemaphoreType.DMA,
        ],
    )
    def edge_kernel(sn_hbm, src_hbm, dst_hbm, w_hbm, z_hbm, out_hbm,
                    idx_s, idx_d, rows_s, rows_d, wv, acc,
                    sem_i, sem_g):
        cid = lax.axis_index("c")
        sid = lax.axis_index("s")
        wid = cid * _NS + sid

        z0 = sid * stripe
        pltpu.sync_copy(z_hbm.at[pl.ds(z0, stripe), :],
                        acc.at[pl.ds(z0, stripe), :])
        pltpu.sync_copy(w_hbm, wv)
        plsc.subcore_barrier()

        base_row = wid * (per_w // 128)

        def chunk_body(kk, carry):
            row0 = base_row + kk * _GRP
            ci1 = pltpu.make_async_copy(src_hbm.at[pl.ds(row0, _GRP), :],
                                        idx_s, sem_i)
            ci2 = pltpu.make_async_copy(dst_hbm.at[pl.ds(row0, _GRP), :],
                                        idx_d, sem_i)
            ci1.start()
            ci2.start()
            ci1.wait()
            ci2.wait()

            gathers = []
            for g in range(_GRP):
                gathers.append(pltpu.make_async_copy(
                    sn_hbm.at[idx_s.at[g]],
                    rows_s.at[pl.ds(g * 128, 128), :], sem_g))
                gathers.append(pltpu.make_async_copy(
                    sn_hbm.at[idx_d.at[g]],
                    rows_d.at[pl.ds(g * 128, 128), :], sem_g))
            for cp in gathers:
                cp.start()
            for cp in gathers:
                cp.wait()

            wvecs = [wv[pl.ds(j * _LANES, _LANES)] for j in range(K)]
            lanes = lax.iota(jnp.int32, _LANES)
            xor_idx = [lanes ^ sh for sh in (8, 4, 2, 1)]

            def edge_body(e, c):
                sr = rows_s[e]
                dr = rows_d[e]
                sb = sr * dr
                for ix in xor_idx:
                    sb = sb + _shuffle(sb, ix)
                p = wvecs[K - 1]
                for j in range(K - 2, -1, -1):
                    p = p * sb + wvecs[j]
                rows_s[e] = p * dr
                rows_d[e] = p * sr
                return c

            lax.fori_loop(0, _CHUNK, edge_body, 0)

            for g in range(_GRP):
                pltpu.sync_copy(rows_s.at[pl.ds(g * 128, 128), :],
                                acc.at[idx_s.at[g]], add=True)
                pltpu.sync_copy(rows_d.at[pl.ds(g * 128, 128), :],
                                acc.at[idx_d.at[g]], add=True)
            return carry

        lax.fori_loop(0, n_chunks, chunk_body, 0)

        plsc.subcore_barrier()
        pltpu.sync_copy(acc.at[pl.ds(z0, stripe), :],
                        out_hbm.at[cid, pl.ds(z0, stripe), :])

    return edge_kernel(sn_pad, src2d, dst2d, wrep, zeros_pad)


def _tc_final(a0, a1, sn, y, mt):
    N, D = sn.shape
    BN = 1000

    def body(a0_ref, a1_ref, sn_ref, y_ref, mt_ref, out_ref):
        a = a0_ref[...] + a1_ref[...]
        sn = sn_ref[...]
        proj = jnp.sum(sn * a, axis=1, keepdims=True)
        out_ref[...] = (-a + sn * proj + y_ref[...]
                        - jnp.dot(sn, mt_ref[...],
                                  preferred_element_type=jnp.float32))

    return pl.pallas_call(
        body,
        grid=(N // BN,),
        in_specs=[
            pl.BlockSpec((BN, D), lambda i: (i, 0)),
            pl.BlockSpec((BN, D), lambda i: (i, 0)),
            pl.BlockSpec((BN, D), lambda i: (i, 0)),
            pl.BlockSpec((BN, D), lambda i: (i, 0)),
            pl.BlockSpec((D, D), lambda i: (0, 0)),
        ],
        out_specs=pl.BlockSpec((BN, D), lambda i: (i, 0)),
        out_shape=jax.ShapeDtypeStruct((N, D), jnp.float32),
    )(a0, a1, sn, y, mt)


def kernel(t, state, ind, omegas, w):
    del t
    N, D = state.shape
    E = ind.shape[0]
    NP = -(-(N + 1) // 128) * 128
    nw = _NC * _NS
    per_w = -(-E // (nw * _CHUNK)) * _CHUNK
    e_pad = per_w * nw

    sn, y, s_sum = _tc_norm_omega(state, omegas)

    pad = jnp.full((e_pad - E,), N, dtype=jnp.int32)
    src2d = jnp.concatenate([ind[:, 0].astype(jnp.int32), pad]
                            ).reshape(e_pad // 128, 128)
    dst2d = jnp.concatenate([ind[:, 1].astype(jnp.int32), pad]
                            ).reshape(e_pad // 128, 128)
    sn_pad = jnp.zeros((NP, D), jnp.float32).at[:N].set(sn)
    wrep = jnp.repeat(w.astype(jnp.float32) * _EPS, _LANES)
    zeros_pad = jnp.zeros((NP, D), jnp.float32)

    acc2 = _sc_edges(sn_pad, src2d, dst2d, wrep, zeros_pad)

    mt = (s_sum.T - s_sum) / (2.0 * N)
    return _tc_final(acc2[0, :N], acc2[1, :N], sn, y, mt)


# speedup vs baseline: 9.6265x; 1.3103x over previous
import functools

import jax
import jax.numpy as jnp
from jax import lax
from jax.experimental import pallas as pl
from jax.experimental.pallas import tpu as pltpu
from jax.experimental.pallas import tpu_sc as plsc

_EPS = 0.1
_GDN = lax.GatherDimensionNumbers(
    offset_dims=(), collapsed_slice_dims=(0,), start_index_map=(0,))


def _shuffle(v, idx):
    return lax.gather(v, idx[:, None], _GDN, slice_sizes=(1,),
                      mode=lax.GatherScatterMode.PROMISE_IN_BOUNDS)
_NC = 2
_NS = 16
_LANES = 16
_CHUNK = 512
_GRP = _CHUNK // 128


def _tc_norm(state):
    N, D = state.shape
    BN = 2000

    def body(st_ref, sn_ref):
        st = st_ref[...]
        inv = lax.rsqrt(jnp.sum(st * st, axis=1, keepdims=True))
        sn_ref[...] = st * inv

    return pl.pallas_call(
        body,
        grid=(N // BN,),
        in_specs=[pl.BlockSpec((BN, D), lambda i: (i, 0))],
        out_specs=pl.BlockSpec((BN, D), lambda i: (i, 0)),
        out_shape=jax.ShapeDtypeStruct((N, D), jnp.float32),
    )(state)


def _tc_omega(sn, omegas):
    N, D = sn.shape
    BN = 1000

    def body(sn_ref, om_ref, y_ref, s_ref):
        sn = sn_ref[...]
        om = om_ref[...]
        t1 = jnp.sum(om * sn[:, None, :], axis=2)
        t2 = jnp.sum(om * sn[:, :, None], axis=1)
        y_ref[...] = 0.5 * (t1 - t2)

        @pl.when(pl.program_id(0) == 0)
        def _():
            s_ref[...] = jnp.zeros_like(s_ref)

        s_ref[...] += jnp.sum(om, axis=0)

    return pl.pallas_call(
        body,
        grid=(N // BN,),
        in_specs=[
            pl.BlockSpec((BN, D), lambda i: (i, 0)),
            pl.BlockSpec((BN, D, D), lambda i: (i, 0, 0)),
        ],
        out_specs=[
            pl.BlockSpec((BN, D), lambda i: (i, 0)),
            pl.BlockSpec((D, D), lambda i: (0, 0)),
        ],
        out_shape=[
            jax.ShapeDtypeStruct((N, D), jnp.float32),
            jax.ShapeDtypeStruct((D, D), jnp.float32),
        ],
    )(sn, omegas)


def _sc_edges(sn_pad, src2d, dst2d, wrep, zeros_pad):
    NP, D = sn_pad.shape
    rows_total = src2d.shape[0]
    e_pad = rows_total * 128
    nw = _NC * _NS
    per_w = e_pad // nw
    n_chunks = per_w // _CHUNK
    stripe = NP // _NS
    K = wrep.shape[0] // _LANES

    mesh = plsc.VectorSubcoreMesh(
        core_axis_name="c", subcore_axis_name="s",
        num_cores=_NC, num_subcores=_NS)

    @functools.partial(
        pl.kernel,
        out_type=jax.ShapeDtypeStruct((_NC, NP, D), jnp.float32),
        mesh=mesh,
        compiler_params=pltpu.CompilerParams(use_tc_tiling_on_sc=False),
        scratch_types=[
            pltpu.VMEM((_GRP, 128), jnp.int32),
            pltpu.VMEM((_GRP, 128), jnp.int32),
            pltpu.VMEM((_CHUNK, D), jnp.float32),
            pltpu.VMEM((_CHUNK, D), jnp.float32),
            pltpu.VMEM((K * _LANES,), jnp.float32),
            pltpu.VMEM_SHARED((NP, D), jnp.float32),
            pltpu.SemaphoreType.DMA,
            pltpu.SemaphoreType.DMA,
        ],
    )
    def edge_kernel(sn_hbm, src_hbm, dst_hbm, w_hbm, z_hbm, out_hbm,
                    idx_s, idx_d, rows_s, rows_d, wv, acc,
                    sem_i, sem_g):
        cid = lax.axis_index("c")
        sid = lax.axis_index("s")
        wid = cid * _NS + sid

        z0 = sid * stripe
        pltpu.sync_copy(z_hbm.at[pl.ds(z0, stripe), :],
                        acc.at[pl.ds(z0, stripe), :])
        pltpu.sync_copy(w_hbm, wv)
        plsc.subcore_barrier()

        base_row = wid * (per_w // 128)

        def chunk_body(kk, carry):
            row0 = base_row + kk * _GRP
            ci1 = pltpu.make_async_copy(src_hbm.at[pl.ds(row0, _GRP), :],
                                        idx_s, sem_i)
            ci2 = pltpu.make_async_copy(dst_hbm.at[pl.ds(row0, _GRP), :],
                                        idx_d, sem_i)
            ci1.start()
            ci2.start()
            ci1.wait()
            ci2.wait()

            gathers = []
            for g in range(_GRP):
                gathers.append(pltpu.make_async_copy(
                    sn_hbm.at[idx_s.at[g]],
                    rows_s.at[pl.ds(g * 128, 128), :], sem_g))
                gathers.append(pltpu.make_async_copy(
                    sn_hbm.at[idx_d.at[g]],
                    rows_d.at[pl.ds(g * 128, 128), :], sem_g))
            for cp in gathers:
                cp.start()
            for cp in gathers:
                cp.wait()

            wvecs = [wv[pl.ds(j * _LANES, _LANES)] for j in range(K)]
            lanes = lax.iota(jnp.int32, _LANES)
            xor_idx = [lanes ^ sh for sh in (8, 4, 2, 1)]

            def edge_body(e, c):
                sr = rows_s[e]
                dr = rows_d[e]
                sb = sr * dr
                for ix in xor_idx:
                    sb = sb + _shuffle(sb, ix)
                p = wvecs[K - 1]
                for j in range(K - 2, -1, -1):
                    p = p * sb + wvecs[j]
                rows_s[e] = p * dr
                rows_d[e] = p * sr
                return c

            lax.fori_loop(0, _CHUNK, edge_body, 0)

            scatters = []
            for g in range(_GRP):
                scatters.append(pltpu.async_copy(
                    rows_s.at[pl.ds(g * 128, 128), :],
                    acc.at[idx_s.at[g]], sem_g, add=True))
                scatters.append(pltpu.async_copy(
                    rows_d.at[pl.ds(g * 128, 128), :],
                    acc.at[idx_d.at[g]], sem_g, add=True))
            for cp in scatters:
                cp.wait()
            return carry

        lax.fori_loop(0, n_chunks, chunk_body, 0)

        plsc.subcore_barrier()
        pltpu.sync_copy(acc.at[pl.ds(z0, stripe), :],
                        out_hbm.at[cid, pl.ds(z0, stripe), :])

    return edge_kernel(sn_pad, src2d, dst2d, wrep, zeros_pad)


def _tc_final(a0, a1, sn, y, mt):
    N, D = sn.shape
    BN = 1000

    def body(a0_ref, a1_ref, sn_ref, y_ref, mt_ref, out_ref):
        a = a0_ref[...] + a1_ref[...]
        sn = sn_ref[...]
        proj = jnp.sum(sn * a, axis=1, keepdims=True)
        out_ref[...] = (-a + sn * proj + y_ref[...]
                        - jnp.dot(sn, mt_ref[...],
                                  preferred_element_type=jnp.float32))

    return pl.pallas_call(
        body,
        grid=(N // BN,),
        in_specs=[
            pl.BlockSpec((BN, D), lambda i: (i, 0)),
            pl.BlockSpec((BN, D), lambda i: (i, 0)),
            pl.BlockSpec((BN, D), lambda i: (i, 0)),
            pl.BlockSpec((BN, D), lambda i: (i, 0)),
            pl.BlockSpec((D, D), lambda i: (0, 0)),
        ],
        out_specs=pl.BlockSpec((BN, D), lambda i: (i, 0)),
        out_shape=jax.ShapeDtypeStruct((N, D), jnp.float32),
    )(a0, a1, sn, y, mt)


def kernel(t, state, ind, omegas, w):
    del t
    N, D = state.shape
    E = ind.shape[0]
    NP = -(-(N + 1) // 128) * 128
    nw = _NC * _NS
    per_w = -(-E // (nw * _CHUNK)) * _CHUNK
    e_pad = per_w * nw

    sn = _tc_norm(state)
    y, s_sum = _tc_omega(sn, omegas)

    pad = jnp.full((e_pad - E,), N, dtype=jnp.int32)
    src2d = jnp.concatenate([ind[:, 0].astype(jnp.int32), pad]
                            ).reshape(e_pad // 128, 128)
    dst2d = jnp.concatenate([ind[:, 1].astype(jnp.int32), pad]
                            ).reshape(e_pad // 128, 128)
    sn_pad = jnp.zeros((NP, D), jnp.float32).at[:N].set(sn)
    wrep = jnp.repeat(w.astype(jnp.float32) * _EPS, _LANES)
    zeros_pad = jnp.zeros((NP, D), jnp.float32)

    acc2 = _sc_edges(sn_pad, src2d, dst2d, wrep, zeros_pad)

    mt = (s_sum.T - s_sum) / (2.0 * N)
    return _tc_final(acc2[0, :N], acc2[1, :N], sn, y, mt)


# speedup vs baseline: 9.9913x; 1.0379x over previous
import functools

import jax
import jax.numpy as jnp
from jax import lax
from jax.experimental import pallas as pl
from jax.experimental.pallas import tpu as pltpu
from jax.experimental.pallas import tpu_sc as plsc

_EPS = 0.1
_GDN = lax.GatherDimensionNumbers(
    offset_dims=(), collapsed_slice_dims=(0,), start_index_map=(0,))


def _shuffle(v, idx):
    return lax.gather(v, idx[:, None], _GDN, slice_sizes=(1,),
                      mode=lax.GatherScatterMode.PROMISE_IN_BOUNDS)
_NC = 2
_NS = 16
_LANES = 16
_CHUNK = 256
_GRP = _CHUNK // 128


def _tc_norm(state):
    N, D = state.shape
    BN = 2000

    def body(st_ref, sn_ref):
        st = st_ref[...]
        inv = lax.rsqrt(jnp.sum(st * st, axis=1, keepdims=True))
        sn_ref[...] = st * inv

    return pl.pallas_call(
        body,
        grid=(N // BN,),
        in_specs=[pl.BlockSpec((BN, D), lambda i: (i, 0))],
        out_specs=pl.BlockSpec((BN, D), lambda i: (i, 0)),
        out_shape=jax.ShapeDtypeStruct((N, D), jnp.float32),
    )(state)


def _tc_omega(sn, omegas):
    N, D = sn.shape
    BN = 1000

    def body(sn_ref, om_ref, y_ref, s_ref):
        sn = sn_ref[...]
        om = om_ref[...]
        t1 = jnp.sum(om * sn[:, None, :], axis=2)
        t2 = jnp.sum(om * sn[:, :, None], axis=1)
        y_ref[...] = 0.5 * (t1 - t2)

        @pl.when(pl.program_id(0) == 0)
        def _():
            s_ref[...] = jnp.zeros_like(s_ref)

        s_ref[...] += jnp.sum(om, axis=0)

    return pl.pallas_call(
        body,
        grid=(N // BN,),
        in_specs=[
            pl.BlockSpec((BN, D), lambda i: (i, 0)),
            pl.BlockSpec((BN, D, D), lambda i: (i, 0, 0)),
        ],
        out_specs=[
            pl.BlockSpec((BN, D), lambda i: (i, 0)),
            pl.BlockSpec((D, D), lambda i: (0, 0)),
        ],
        out_shape=[
            jax.ShapeDtypeStruct((N, D), jnp.float32),
            jax.ShapeDtypeStruct((D, D), jnp.float32),
        ],
    )(sn, omegas)


def _sc_edges(sn_pad, src2d, dst2d, wrep, zeros_pad):
    NP, D = sn_pad.shape
    rows_total = src2d.shape[0]
    e_pad = rows_total * 128
    nw = _NC * _NS
    per_w = e_pad // nw
    n_chunks = per_w // _CHUNK
    stripe = NP // _NS
    K = wrep.shape[0] // _LANES

    mesh = plsc.VectorSubcoreMesh(
        core_axis_name="c", subcore_axis_name="s",
        num_cores=_NC, num_subcores=_NS)

    @functools.partial(
        pl.kernel,
        out_type=jax.ShapeDtypeStruct((_NC, NP, D), jnp.float32),
        mesh=mesh,
        compiler_params=pltpu.CompilerParams(use_tc_tiling_on_sc=False),
        scratch_types=[
            pltpu.VMEM((_GRP, 128), jnp.int32),
            pltpu.VMEM((_GRP, 128), jnp.int32),
            pltpu.VMEM((_GRP, 128), jnp.int32),
            pltpu.VMEM((_GRP, 128), jnp.int32),
            pltpu.VMEM((_CHUNK, D), jnp.float32),
            pltpu.VMEM((_CHUNK, D), jnp.float32),
            pltpu.VMEM((_CHUNK, D), jnp.float32),
            pltpu.VMEM((_CHUNK, D), jnp.float32),
            pltpu.VMEM((K * _LANES,), jnp.float32),
            pltpu.VMEM_SHARED((NP, D), jnp.float32),
            pltpu.SemaphoreType.DMA,
            pltpu.SemaphoreType.DMA,
            pltpu.SemaphoreType.DMA,
            pltpu.SemaphoreType.DMA,
        ],
    )
    def edge_kernel(sn_hbm, src_hbm, dst_hbm, w_hbm, z_hbm, out_hbm,
                    idx_s0, idx_d0, idx_s1, idx_d1,
                    rows_s0, rows_d0, rows_s1, rows_d1, wv, acc,
                    sem_g0, sem_g1, sem_s0, sem_s1):
        cid = lax.axis_index("c")
        sid = lax.axis_index("s")
        wid = cid * _NS + sid
        idx_bufs = ((idx_s0, idx_d0), (idx_s1, idx_d1))
        row_bufs = ((rows_s0, rows_d0), (rows_s1, rows_d1))
        sem_g = (sem_g0, sem_g1)
        sem_s = (sem_s0, sem_s1)

        z0 = sid * stripe
        pltpu.sync_copy(z_hbm.at[pl.ds(z0, stripe), :],
                        acc.at[pl.ds(z0, stripe), :])
        pltpu.sync_copy(w_hbm, wv)
        plsc.subcore_barrier()

        base_row = wid * (per_w // 128)

        def idx_copy(b, ck):
            row0 = base_row + ck * _GRP
            pltpu.sync_copy(src_hbm.at[pl.ds(row0, _GRP), :], idx_bufs[b][0])
            pltpu.sync_copy(dst_hbm.at[pl.ds(row0, _GRP), :], idx_bufs[b][1])

        def gather_descs(b):
            (isb, idb), (rsb, rdb) = idx_bufs[b], row_bufs[b]
            ds_ = []
            for g in range(_GRP):
                ds_.append(pltpu.make_async_copy(
                    sn_hbm.at[isb.at[g]],
                    rsb.at[pl.ds(g * 128, 128), :], sem_g[b]))
                ds_.append(pltpu.make_async_copy(
                    sn_hbm.at[idb.at[g]],
                    rdb.at[pl.ds(g * 128, 128), :], sem_g[b]))
            return ds_

        def scatter_drain(b):
            (isb, idb), (rsb, rdb) = idx_bufs[b], row_bufs[b]
            for g in range(_GRP):
                pltpu.make_async_copy(rsb.at[pl.ds(g * 128, 128), :],
                                      acc.at[isb.at[g]], sem_s[b]).wait()
                pltpu.make_async_copy(rdb.at[pl.ds(g * 128, 128), :],
                                      acc.at[idb.at[g]], sem_s[b]).wait()

        wvecs = [wv[pl.ds(j * _LANES, _LANES)] for j in range(K)]
        lanes = lax.iota(jnp.int32, _LANES)
        xor_idx = [lanes ^ sh for sh in (8, 4, 2, 1)]

        def compute(b):
            rsb, rdb = row_bufs[b]

            def edge_body(e, c):
                sr = rsb[e]
                dr = rdb[e]
                sb = sr * dr
                for ix in xor_idx:
                    sb = sb + _shuffle(sb, ix)
                p = wvecs[K - 1]
                for j in range(K - 2, -1, -1):
                    p = p * sb + wvecs[j]
                rsb[e] = p * dr
                rdb[e] = p * sr
                return c

            lax.fori_loop(0, _CHUNK, edge_body, 0)

        idx_copy(0, 0)
        for cp in gather_descs(0):
            cp.start()

        def pair_body(k2, carry):
            for b in (0, 1):
                ck = k2 * 2 + b
                nb = 1 - b

                @pl.when(ck >= 1)
                def _():
                    scatter_drain(nb)

                @pl.when(ck + 1 < n_chunks)
                def _():
                    idx_copy(nb, ck + 1)
                    for cp in gather_descs(nb):
                        cp.start()

                for cp in gather_descs(b):
                    cp.wait()
                compute(b)

                (isb, idb), (rsb, rdb) = idx_bufs[b], row_bufs[b]
                for g in range(_GRP):
                    pltpu.async_copy(rsb.at[pl.ds(g * 128, 128), :],
                                     acc.at[isb.at[g]], sem_s[b], add=True)
                    pltpu.async_copy(rdb.at[pl.ds(g * 128, 128), :],
                                     acc.at[idb.at[g]], sem_s[b], add=True)
            return carry

        lax.fori_loop(0, n_chunks // 2, pair_body, 0)
        scatter_drain(1)

        plsc.subcore_barrier()
        pltpu.sync_copy(acc.at[pl.ds(z0, stripe), :],
                        out_hbm.at[cid, pl.ds(z0, stripe), :])

    return edge_kernel(sn_pad, src2d, dst2d, wrep, zeros_pad)


def _tc_final(a0, a1, sn, y, mt):
    N, D = sn.shape
    BN = 1000

    def body(a0_ref, a1_ref, sn_ref, y_ref, mt_ref, out_ref):
        a = a0_ref[...] + a1_ref[...]
        sn = sn_ref[...]
        proj = jnp.sum(sn * a, axis=1, keepdims=True)
        out_ref[...] = (-a + sn * proj + y_ref[...]
                        - jnp.dot(sn, mt_ref[...],
                                  preferred_element_type=jnp.float32))

    return pl.pallas_call(
        body,
        grid=(N // BN,),
        in_specs=[
            pl.BlockSpec((BN, D), lambda i: (i, 0)),
            pl.BlockSpec((BN, D), lambda i: (i, 0)),
            pl.BlockSpec((BN, D), lambda i: (i, 0)),
            pl.BlockSpec((BN, D), lambda i: (i, 0)),
            pl.BlockSpec((D, D), lambda i: (0, 0)),
        ],
        out_specs=pl.BlockSpec((BN, D), lambda i: (i, 0)),
        out_shape=jax.ShapeDtypeStruct((N, D), jnp.float32),
    )(a0, a1, sn, y, mt)


def kernel(t, state, ind, omegas, w):
    del t
    N, D = state.shape
    E = ind.shape[0]
    NP = -(-(N + 1) // 128) * 128
    nw = _NC * _NS
    per_w = -(-E // (nw * _CHUNK * 2)) * (_CHUNK * 2)
    e_pad = per_w * nw

    sn = _tc_norm(state)
    y, s_sum = _tc_omega(sn, omegas)

    pad = jnp.full((e_pad - E,), N, dtype=jnp.int32)
    src2d = jnp.concatenate([ind[:, 0].astype(jnp.int32), pad]
                            ).reshape(e_pad // 128, 128)
    dst2d = jnp.concatenate([ind[:, 1].astype(jnp.int32), pad]
                            ).reshape(e_pad // 128, 128)
    sn_pad = jnp.zeros((NP, D), jnp.float32).at[:N].set(sn)
    wrep = jnp.repeat(w.astype(jnp.float32) * _EPS, _LANES)
    zeros_pad = jnp.zeros((NP, D), jnp.float32)

    acc2 = _sc_edges(sn_pad, src2d, dst2d, wrep, zeros_pad)

    mt = (s_sum.T - s_sum) / (2.0 * N)
    return _tc_final(acc2[0, :N], acc2[1, :N], sn, y, mt)
